# baseline (speedup 1.0000x reference)
import functools

import jax
import jax.numpy as jnp
from jax import lax
from jax.experimental import pallas as pl
from jax.experimental.pallas import tpu as pltpu
from jax.experimental.pallas import tpu_sc as plsc

N = 10000
E = 320000
DIN = 128
D = 16
NC = 2
NS = 16
NW = NC * NS
CH = 2000
EPW = E // NW
NCH = EPW // CH
RPT = 640
NP = NS * RPT
BLK = 1024
PRB = BLK * D // 128
PR = NP * D // 128

_mesh = plsc.VectorSubcoreMesh(
    core_axis_name="c", subcore_axis_name="s", num_cores=NC, num_subcores=NS)


@functools.partial(
    pl.kernel,
    out_type=jax.ShapeDtypeStruct((NC, NP), jnp.float32),
    mesh=_mesh,
    scratch_types=[
        pltpu.VMEM((EPW,), jnp.int32),
        pltpu.VMEM((EPW,), jnp.float32),
        pltpu.VMEM((RPT,), jnp.float32),
        pltpu.VMEM_SHARED((NP,), jnp.float32),
        pltpu.SemaphoreType.DMA,
    ],
)
def _deg_kernel(col1, ew1, out, cidx_v, ew_v, zbuf, deg_s, sem):
    c = lax.axis_index("c")
    s = lax.axis_index("s")
    wid = s * NC + c

    def _z(i, carry):
        zbuf[pl.ds(i * 16, 16)] = jnp.zeros((16,), jnp.float32)
        return carry

    lax.fori_loop(0, RPT // 16, _z, 0)
    pltpu.sync_copy(zbuf, deg_s.at[pl.ds(s * RPT, RPT)])
    pltpu.sync_copy(col1.at[pl.ds(wid * EPW, EPW)], cidx_v)
    pltpu.sync_copy(ew1.at[pl.ds(wid * EPW, EPW)], ew_v)
    plsc.subcore_barrier()
    pltpu.async_copy(ew_v, deg_s.at[cidx_v], sem, add=True).wait()
    plsc.subcore_barrier()
    pltpu.sync_copy(deg_s.at[pl.ds(s * RPT, RPT)],
                    out.at[c, pl.ds(s * RPT, RPT)])


_PROP_SCRATCH = [
    pltpu.VMEM((4, CH), jnp.int32),
    pltpu.VMEM((4, CH), jnp.int32),
    pltpu.VMEM((4, CH), jnp.float32),
    pltpu.VMEM((3, CH, D), jnp.float32),
    pltpu.VMEM_SHARED((NP, D), jnp.float32),
    pltpu.SemaphoreType.DMA,
    pltpu.SemaphoreType.DMA,
    pltpu.SemaphoreType.DMA,
    pltpu.SemaphoreType.DMA,
    pltpu.SemaphoreType.DMA,
    pltpu.SemaphoreType.DMA,
    pltpu.SemaphoreType.DMA,
    pltpu.SemaphoreType.DMA,
    pltpu.SemaphoreType.DMA,
    pltpu.SemaphoreType.DMA,
]


def _edge_pipeline(hsrc, row1, col1, ew1, out, ridx_v, cidx_v, ew_v, rows_v,
                   acc_s, gsem, ssem, lsem, c, s, wid):
    ldescs, gdescs, sdescs = {}, {}, {}

    def _issue_l(k):
        b = k % 4
        eb = wid * EPW + k * CH
        ldescs[k] = [
            pltpu.async_copy(row1.at[pl.ds(eb, CH)], ridx_v.at[b], lsem[b]),
            pltpu.async_copy(col1.at[pl.ds(eb, CH)], cidx_v.at[b], lsem[b]),
            pltpu.async_copy(ew1.at[pl.ds(eb, CH)], ew_v.at[b], lsem[b]),
        ]

    def _issue_g(k):
        b4, b3 = k % 4, k % 3
        gdescs[k] = [
            pltpu.async_copy(hsrc.at[ridx_v.at[b4]], rows_v.at[b3], gsem[b3])
        ]

    def _issue_s(k):
        b4, b3 = k % 4, k % 3
        sdescs[k] = [
            pltpu.async_copy(rows_v.at[b3], acc_s.at[cidx_v.at[b4]],
                             ssem[b3], add=True)
        ]

    def _drain(descs):
        for d_ in descs:
            d_.wait()

    _issue_l(0)
    _issue_l(1)
    _drain(ldescs[0])
    _issue_g(0)
    plsc.subcore_barrier()

    for k in range(NCH):
        b4, b3 = k % 4, k % 3
        if k >= 2:
            _drain(sdescs[k - 2])
        if k + 1 < NCH:
            _drain(ldescs[k + 1])
            _issue_g(k + 1)
        if k + 2 < NCH:
            _issue_l(k + 2)
        _drain(gdescs[k])

        def _m(g, carry2, b4=b4, b3=b3):
            ew16 = ew_v[b4, pl.ds(g * 16, 16)]
            base = g * 16
            for e in range(16):
                rows_v[b3, base + e, :] = rows_v[b3, base + e, :] * ew16[e]
            return carry2

        lax.fori_loop(0, CH // 16, _m, 0)
        _issue_s(k)

    _drain(sdescs[NCH - 2])
    _drain(sdescs[NCH - 1])
    plsc.subcore_barrier()
    pltpu.sync_copy(acc_s.at[pl.ds(s * RPT, RPT)],
                    out.at[c, pl.ds(s * RPT, RPT)])


@functools.partial(
    pl.kernel,
    out_type=jax.ShapeDtypeStruct((NC, NP, D), jnp.float32),
    mesh=_mesh,
    scratch_types=_PROP_SCRATCH,
    compiler_params=pltpu.CompilerParams(use_tc_tiling_on_sc=False),
)
def _prop1_kernel(h, row1, col1, ew1, out, ridx_v, cidx_v, ew_v, rows_v,
                  acc_s, g0, g1, g2, s0, s1, s2, l0, l1, l2, l3):
    c = lax.axis_index("c")
    s = lax.axis_index("s")
    wid = s * NC + c

    @pl.when(c == 0)
    def _seed():
        pltpu.sync_copy(h.at[pl.ds(s * RPT, RPT)],
                        acc_s.at[pl.ds(s * RPT, RPT)])

    @pl.when(c != 0)
    def _zero():
        def _z(i, carry):
            rows_v[0, i, :] = jnp.zeros((D,), jnp.float32)
            return carry

        lax.fori_loop(0, RPT, _z, 0)
        pltpu.sync_copy(rows_v.at[0, pl.ds(0, RPT)],
                        acc_s.at[pl.ds(s * RPT, RPT)])

    _edge_pipeline(h, row1, col1, ew1, out, ridx_v, cidx_v, ew_v, rows_v,
                   acc_s, (g0, g1, g2), (s0, s1, s2), (l0, l1, l2, l3),
                   c, s, wid)


@functools.partial(
    pl.kernel,
    out_type=jax.ShapeDtypeStruct((NP, D), jnp.float32),
    mesh=_mesh,
    scratch_types=[
        pltpu.VMEM((2 * RPT, D), jnp.float32),
        pltpu.VMEM((RPT, D), jnp.float32),
        pltpu.VMEM((RPT,), jnp.float32),
        pltpu.VMEM((D, D), jnp.float32),
        pltpu.VMEM((D,), jnp.float32),
    ],
    compiler_params=pltpu.CompilerParams(use_tc_tiling_on_sc=False),
)
def _mix_kernel(p1, dis, w2, b1, out, p_v, h2_v, dis_v, w2_v, b1_v):
    c = lax.axis_index("c")
    s = lax.axis_index("s")

    @pl.when(c == 0)
    def _go():
        base = s * RPT
        pltpu.sync_copy(p1.at[0, pl.ds(base, RPT)], p_v.at[pl.ds(0, RPT)])
        pltpu.sync_copy(p1.at[1, pl.ds(base, RPT)], p_v.at[pl.ds(RPT, RPT)])
        pltpu.sync_copy(dis.at[pl.ds(base, RPT)], dis_v)
        pltpu.sync_copy(w2, w2_v)
        pltpu.sync_copy(b1, b1_v)
        w2rows = [w2_v[k, :] for k in range(D)]
        b1vec = b1_v[...]

        def _row(g, carry):
            d16 = dis_v[pl.ds(g * 16, 16)]
            for e in range(16):
                r = g * 16 + e
                srow = p_v[r, :] + p_v[RPT + r, :]
                a = jnp.maximum(d16[e] * srow + b1vec, 0.0)
                acc = a[0] * w2rows[0]
                for k in range(1, D):
                    acc = acc + a[k] * w2rows[k]
                h2_v[r, :] = d16[e] * acc
            return carry

        lax.fori_loop(0, RPT // 16, _row, 0)
        pltpu.sync_copy(h2_v, out.at[pl.ds(base, RPT)])


ESPLIT = E // 10


def _split_body(ei_ref, row_ref, col_ref):
    row_ref[...] = ei_ref[0, :]
    col_ref[...] = ei_ref[1, :]


_split = pl.pallas_call(
    _split_body,
    out_shape=(jax.ShapeDtypeStruct((E,), jnp.int32),
               jax.ShapeDtypeStruct((E,), jnp.int32)),
)


def _tcdis_body(degp_ref, dis_ref):
    dis_ref[...] = lax.rsqrt(degp_ref[0, :] + degp_ref[1, :] + 1.0)


_tcdis = pl.pallas_call(
    _tcdis_body,
    out_shape=jax.ShapeDtypeStruct((NP,), jnp.float32),
)


def _tc1_body(x_ref, w1_ref, dis_ref, h1p_ref):
    h1p_ref[...] = dis_ref[...][:, None] * jnp.dot(
        x_ref[...], w1_ref[...], preferred_element_type=jnp.float32)


BLK1 = 512

_tc1 = pl.pallas_call(
    _tc1_body,
    grid=(NP // BLK1,),
    in_specs=[
        pl.BlockSpec((BLK1, DIN), lambda i: (i, 0)),
        pl.BlockSpec((DIN, D), lambda i: (0, 0)),
        pl.BlockSpec((BLK1,), lambda i: (i,)),
    ],
    out_specs=pl.BlockSpec((BLK1, D), lambda i: (i, 0)),
    out_shape=jax.ShapeDtypeStruct((NP, D), jnp.float32),
)


def _tc3_body(p2_ref, dis_ref, b2_ref, out_ref):
    z = dis_ref[...][:, None] * (p2_ref[0] + p2_ref[1]) + b2_ref[...]
    m = jnp.max(z, axis=1, keepdims=True)
    lse = jnp.log(jnp.sum(jnp.exp(z - m), axis=1, keepdims=True)) + m
    out_ref[...] = z - lse


_tc3 = pl.pallas_call(
    _tc3_body,
    grid=(NP // BLK,),
    in_specs=[
        pl.BlockSpec((NC, BLK, D), lambda i: (0, i, 0)),
        pl.BlockSpec((BLK,), lambda i: (i,)),
        pl.BlockSpec((1, D), lambda i: (0, 0)),
    ],
    out_specs=pl.BlockSpec((BLK, D), lambda i: (i, 0)),
    out_shape=jax.ShapeDtypeStruct((NP, D), jnp.float32),
)


def kernel(x, edge_index, edge_weight, W1, b1, W2, b2):
    row1, col1 = _split(edge_index.astype(jnp.int32))
    xp = jnp.pad(x, ((0, NP - N), (0, 0)))

    degp = _deg_kernel(col1, edge_weight)
    dis = _tcdis(degp)
    h1p = _tc1(xp, W1, dis)

    p1 = _prop1_kernel(h1p, row1, col1, edge_weight)
    h2 = _mix_kernel(p1, dis, W2, b1)
    p2 = _prop1_kernel(h2, row1, col1, edge_weight)
    out = _tc3(p2, dis, b2.reshape(1, D))
    return out[:N]


# speedup vs baseline: 1.0339x; 1.0339x over previous
import functools

import jax
import jax.numpy as jnp
from jax import lax
from jax.experimental import pallas as pl
from jax.experimental.pallas import tpu as pltpu
from jax.experimental.pallas import tpu_sc as plsc

N = 10000
E = 320000
DIN = 128
D = 16
NC = 2
NS = 16
NW = NC * NS
CH = 2000
EPW = E // NW
NCH = EPW // CH
RPT = 640
NP = NS * RPT
BLK = 1024
PRB = BLK * D // 128
PR = NP * D // 128

_mesh = plsc.VectorSubcoreMesh(
    core_axis_name="c", subcore_axis_name="s", num_cores=NC, num_subcores=NS)


@functools.partial(
    pl.kernel,
    out_type=jax.ShapeDtypeStruct((NC, NP), jnp.float32),
    mesh=_mesh,
    scratch_types=[
        pltpu.VMEM((EPW,), jnp.int32),
        pltpu.VMEM((EPW,), jnp.float32),
        pltpu.VMEM((RPT,), jnp.float32),
        pltpu.VMEM_SHARED((NP,), jnp.float32),
        pltpu.SemaphoreType.DMA,
    ],
)
def _deg_kernel(col1, ew1, out, cidx_v, ew_v, zbuf, deg_s, sem):
    c = lax.axis_index("c")
    s = lax.axis_index("s")
    wid = s * NC + c

    def _z(i, carry):
        zbuf[pl.ds(i * 16, 16)] = jnp.zeros((16,), jnp.float32)
        return carry

    lax.fori_loop(0, RPT // 16, _z, 0)
    pltpu.sync_copy(zbuf, deg_s.at[pl.ds(s * RPT, RPT)])
    pltpu.sync_copy(col1.at[pl.ds(wid * EPW, EPW)], cidx_v)
    pltpu.sync_copy(ew1.at[pl.ds(wid * EPW, EPW)], ew_v)
    plsc.subcore_barrier()
    pltpu.async_copy(ew_v, deg_s.at[cidx_v], sem, add=True).wait()
    plsc.subcore_barrier()
    pltpu.sync_copy(deg_s.at[pl.ds(s * RPT, RPT)],
                    out.at[c, pl.ds(s * RPT, RPT)])


_PROP_SCRATCH = [
    pltpu.VMEM((4, CH), jnp.int32),
    pltpu.VMEM((4, CH), jnp.int32),
    pltpu.VMEM((4, CH), jnp.float32),
    pltpu.VMEM((3, CH, D), jnp.float32),
    pltpu.VMEM_SHARED((NP, D), jnp.float32),
    pltpu.SemaphoreType.DMA,
    pltpu.SemaphoreType.DMA,
    pltpu.SemaphoreType.DMA,
    pltpu.SemaphoreType.DMA,
    pltpu.SemaphoreType.DMA,
    pltpu.SemaphoreType.DMA,
    pltpu.SemaphoreType.DMA,
    pltpu.SemaphoreType.DMA,
    pltpu.SemaphoreType.DMA,
    pltpu.SemaphoreType.DMA,
]


def _edge_pipeline(hsrc, row1, col1, ew1, out, ridx_v, cidx_v, ew_v, rows_v,
                   acc_s, gsem, ssem, lsem, c, s, wid):
    ldescs, gdescs, sdescs = {}, {}, {}

    def _issue_l(k):
        b = k % 4
        eb = wid * EPW + k * CH
        ldescs[k] = [
            pltpu.async_copy(row1.at[pl.ds(eb, CH)], ridx_v.at[b], lsem[b]),
            pltpu.async_copy(col1.at[pl.ds(eb, CH)], cidx_v.at[b], lsem[b]),
            pltpu.async_copy(ew1.at[pl.ds(eb, CH)], ew_v.at[b], lsem[b]),
        ]

    def _issue_g(k):
        b4, b3 = k % 4, k % 3
        gdescs[k] = [
            pltpu.async_copy(hsrc.at[ridx_v.at[b4]], rows_v.at[b3], gsem[b3])
        ]

    def _issue_s(k):
        b4, b3 = k % 4, k % 3
        sdescs[k] = [
            pltpu.async_copy(rows_v.at[b3], acc_s.at[cidx_v.at[b4]],
                             ssem[b3], add=True)
        ]

    def _drain(descs):
        for d_ in descs:
            d_.wait()

    _issue_l(0)
    _issue_l(1)
    _drain(ldescs[0])
    _issue_g(0)
    plsc.subcore_barrier()

    for k in range(NCH):
        b4, b3 = k % 4, k % 3
        if k >= 2:
            _drain(sdescs[k - 2])
        if k + 1 < NCH:
            _drain(ldescs[k + 1])
            _issue_g(k + 1)
        if k + 2 < NCH:
            _issue_l(k + 2)
        _drain(gdescs[k])

        def _m(g, carry2, b4=b4, b3=b3):
            ew16 = ew_v[b4, pl.ds(g * 16, 16)]
            base = g * 16
            for e in range(16):
                rows_v[b3, base + e, :] = rows_v[b3, base + e, :] * ew16[e]
            return carry2

        lax.fori_loop(0, CH // 16, _m, 0)
        _issue_s(k)

    _drain(sdescs[NCH - 2])
    _drain(sdescs[NCH - 1])
    plsc.subcore_barrier()
    pltpu.sync_copy(acc_s.at[pl.ds(s * RPT, RPT)],
                    out.at[c, pl.ds(s * RPT, RPT)])


@functools.partial(
    pl.kernel,
    out_type=jax.ShapeDtypeStruct((NC, NP, D), jnp.float32),
    mesh=_mesh,
    scratch_types=_PROP_SCRATCH,
    compiler_params=pltpu.CompilerParams(use_tc_tiling_on_sc=False),
)
def _prop1_kernel(h, row1, col1, ew1, out, ridx_v, cidx_v, ew_v, rows_v,
                  acc_s, g0, g1, g2, s0, s1, s2, l0, l1, l2, l3):
    c = lax.axis_index("c")
    s = lax.axis_index("s")
    wid = s * NC + c

    @pl.when(c == 0)
    def _seed():
        pltpu.sync_copy(h.at[pl.ds(s * RPT, RPT)],
                        acc_s.at[pl.ds(s * RPT, RPT)])

    @pl.when(c != 0)
    def _zero():
        def _z(i, carry):
            rows_v[0, i, :] = jnp.zeros((D,), jnp.float32)
            return carry

        lax.fori_loop(0, RPT, _z, 0)
        pltpu.sync_copy(rows_v.at[0, pl.ds(0, RPT)],
                        acc_s.at[pl.ds(s * RPT, RPT)])

    _edge_pipeline(h, row1, col1, ew1, out, ridx_v, cidx_v, ew_v, rows_v,
                   acc_s, (g0, g1, g2), (s0, s1, s2), (l0, l1, l2, l3),
                   c, s, wid)


@functools.partial(
    pl.kernel,
    out_type=jax.ShapeDtypeStruct((NP, D), jnp.float32),
    mesh=_mesh,
    scratch_types=[
        pltpu.VMEM((2 * RPT, D), jnp.float32),
        pltpu.VMEM((RPT, D), jnp.float32),
        pltpu.VMEM((RPT,), jnp.float32),
        pltpu.VMEM((D, D), jnp.float32),
        pltpu.VMEM((D,), jnp.float32),
    ],
    compiler_params=pltpu.CompilerParams(use_tc_tiling_on_sc=False),
)
def _mix_kernel(p1, dis, w2, b1, out, p_v, h2_v, dis_v, w2_v, b1_v):
    c = lax.axis_index("c")
    s = lax.axis_index("s")

    @pl.when(c == 0)
    def _go():
        base = s * RPT
        pltpu.sync_copy(p1.at[0, pl.ds(base, RPT)], p_v.at[pl.ds(0, RPT)])
        pltpu.sync_copy(p1.at[1, pl.ds(base, RPT)], p_v.at[pl.ds(RPT, RPT)])
        pltpu.sync_copy(dis.at[pl.ds(base, RPT)], dis_v)
        pltpu.sync_copy(w2, w2_v)
        pltpu.sync_copy(b1, b1_v)
        w2rows = [w2_v[k, :] for k in range(D)]
        b1vec = b1_v[...]

        def _row(g, carry):
            d16 = dis_v[pl.ds(g * 16, 16)]
            for e in range(16):
                r = g * 16 + e
                srow = p_v[r, :] + p_v[RPT + r, :]
                a = jnp.maximum(d16[e] * srow + b1vec, 0.0)
                acc = a[0] * w2rows[0]
                for k in range(1, D):
                    acc = acc + a[k] * w2rows[k]
                h2_v[r, :] = d16[e] * acc
            return carry

        lax.fori_loop(0, RPT // 16, _row, 0)
        pltpu.sync_copy(h2_v, out.at[pl.ds(base, RPT)])


ESPLIT = E // 10


def _split_body(ei_ref, row_ref, col_ref):
    row_ref[...] = ei_ref[0, :]
    col_ref[...] = ei_ref[1, :]


_split = pl.pallas_call(
    _split_body,
    out_shape=(jax.ShapeDtypeStruct((E,), jnp.int32),
               jax.ShapeDtypeStruct((E,), jnp.int32)),
)


def _tcdis_body(degp_ref, dis_ref):
    dis_ref[...] = lax.rsqrt(degp_ref[0, :] + degp_ref[1, :] + 1.0)


_tcdis = pl.pallas_call(
    _tcdis_body,
    out_shape=jax.ShapeDtypeStruct((NP,), jnp.float32),
)


def _tc1_body(x_ref, w1_ref, dis_ref, h1p_ref):
    h1p_ref[...] = dis_ref[...][:, None] * jnp.dot(
        x_ref[...], w1_ref[...], preferred_element_type=jnp.float32)


_tc1 = pl.pallas_call(
    _tc1_body,
    grid=(NP // BLK,),
    in_specs=[
        pl.BlockSpec((BLK, DIN), lambda i: (i, 0)),
        pl.BlockSpec((DIN, D), lambda i: (0, 0)),
        pl.BlockSpec((BLK,), lambda i: (i,)),
    ],
    out_specs=pl.BlockSpec((BLK, D), lambda i: (i, 0)),
    out_shape=jax.ShapeDtypeStruct((NP, D), jnp.float32),
)


def _tc3_body(p2_ref, dis_ref, b2_ref, out_ref):
    z = dis_ref[...][:, None] * (p2_ref[0] + p2_ref[1]) + b2_ref[...]
    m = jnp.max(z, axis=1, keepdims=True)
    lse = jnp.log(jnp.sum(jnp.exp(z - m), axis=1, keepdims=True)) + m
    out_ref[...] = z - lse


_tc3 = pl.pallas_call(
    _tc3_body,
    grid=(NP // BLK,),
    in_specs=[
        pl.BlockSpec((NC, BLK, D), lambda i: (0, i, 0)),
        pl.BlockSpec((BLK,), lambda i: (i,)),
        pl.BlockSpec((1, D), lambda i: (0, 0)),
    ],
    out_specs=pl.BlockSpec((BLK, D), lambda i: (i, 0)),
    out_shape=jax.ShapeDtypeStruct((NP, D), jnp.float32),
)


def kernel(x, edge_index, edge_weight, W1, b1, W2, b2):
    row1, col1 = _split(edge_index.astype(jnp.int32))
    xp = jnp.pad(x, ((0, NP - N), (0, 0)))

    degp = _deg_kernel(col1, edge_weight)
    dis = _tcdis(degp)
    h1p = _tc1(xp, W1, dis)

    p1 = _prop1_kernel(h1p, row1, col1, edge_weight)
    h2 = _mix_kernel(p1, dis, W2, b1)
    p2 = _prop1_kernel(h2, row1, col1, edge_weight)
    out = _tc3(p2, dis, b2.reshape(1, D))
    return out[:N]
